# baseline (speedup 1.0000x reference)
import functools

import jax
import jax.numpy as jnp
from jax import lax
from jax.experimental import pallas as pl
from jax.experimental.pallas import tpu as pltpu
from jax.experimental.pallas import tpu_sc as plsc

_NUM_CORES = 2
_NUM_SUBCORES = 16
_NUM_WORKERS = _NUM_CORES * _NUM_SUBCORES

_BATCH = 16384
_DIM = 64
_VOCAB = 1000001

_CHUNK = 128
_WORDS_PER_W = _BATCH * _DIM // _NUM_WORKERS
_N_CHUNKS = _WORDS_PER_W // _CHUNK
_WAVE = 8
_N_WAVES = _N_CHUNKS // _WAVE


def _gather_body(widx_hbm, tflat_hbm, out_hbm, idx_v, dst_v, sem):
    wid = lax.axis_index("s") * _NUM_CORES + lax.axis_index("c")
    pltpu.sync_copy(widx_hbm.at[wid], idx_v)

    def fire(w):
        for k in range(_WAVE):
            row = w * _WAVE + k
            pltpu.async_copy(tflat_hbm.at[idx_v.at[row]], dst_v.at[row], sem)

    fire(0)

    def body(w, carry):
        @pl.when(w < _N_WAVES - 1)
        def _():
            fire(w + 1)

        for _ in range(_WAVE):
            pltpu.make_async_copy(
                tflat_hbm.at[pl.ds(0, _CHUNK)], dst_v.at[0], sem
            ).wait()
        return carry

    lax.fori_loop(0, _N_WAVES, body, 0)
    pltpu.sync_copy(dst_v, out_hbm.at[wid])


@jax.jit
def _sc_gather(widx, tflat):
    mesh = plsc.VectorSubcoreMesh(core_axis_name="c", subcore_axis_name="s")
    call = functools.partial(
        pl.kernel,
        mesh=mesh,
        out_type=jax.ShapeDtypeStruct(
            (_NUM_WORKERS, _N_CHUNKS, _CHUNK), jnp.float32
        ),
        scratch_types=[
            pltpu.VMEM((_N_CHUNKS, _CHUNK), jnp.int32),
            pltpu.VMEM((_N_CHUNKS, _CHUNK), jnp.float32),
            pltpu.SemaphoreType.DMA,
        ],
        compiler_params=pltpu.CompilerParams(use_tc_tiling_on_sc=False),
    )(_gather_body)
    return call(widx, tflat)


def kernel(indices, table):
    idx = indices.astype(jnp.int32)
    tflat = table.T.reshape(-1)
    widx = idx[:, None] + (jnp.arange(_DIM, dtype=jnp.int32) * _VOCAB)[None, :]
    widx = widx.reshape(_NUM_WORKERS, _N_CHUNKS, _CHUNK)
    out = _sc_gather(widx, tflat)
    return out.reshape(_BATCH, _DIM)


# speedup vs baseline: 9.0703x; 9.0703x over previous
import functools

import jax
import jax.numpy as jnp
from jax import lax
from jax.experimental import pallas as pl
from jax.experimental.pallas import tpu as pltpu
from jax.experimental.pallas import tpu_sc as plsc

_NUM_CORES = 2
_NUM_SUBCORES = 16
_NUM_WORKERS = _NUM_CORES * _NUM_SUBCORES

_BATCH = 16384
_DIM = 64
_VOCAB = 1000001
_VPAD = 1000008
_WPAD = 128

_CHUNK = 64
_ROWS_PER_W = _BATCH // _NUM_WORKERS
_N_CHUNKS = _ROWS_PER_W // _CHUNK


def _gather_body(idx_hbm, tpad_hbm, out_hbm, idx_v, dst_v, sem):
    wid = lax.axis_index("s") * _NUM_CORES + lax.axis_index("c")
    pltpu.sync_copy(idx_hbm.at[wid], idx_v)
    copies = [
        pltpu.async_copy(tpad_hbm.at[idx_v.at[j]], dst_v.at[j], sem)
        for j in range(_N_CHUNKS)
    ]
    for c in copies:
        c.wait()
    pltpu.sync_copy(dst_v, out_hbm.at[wid])


@jax.jit
def _sc_gather(idx, tpad):
    mesh = plsc.VectorSubcoreMesh(core_axis_name="c", subcore_axis_name="s")
    call = functools.partial(
        pl.kernel,
        mesh=mesh,
        out_type=jax.ShapeDtypeStruct(
            (_NUM_WORKERS, _N_CHUNKS, _CHUNK, _WPAD), jnp.float32
        ),
        scratch_types=[
            pltpu.VMEM((_N_CHUNKS, _CHUNK), jnp.int32),
            pltpu.VMEM((_N_CHUNKS, _CHUNK, _WPAD), jnp.float32),
            pltpu.SemaphoreType.DMA,
        ],
        compiler_params=pltpu.CompilerParams(use_tc_tiling_on_sc=True),
    )(_gather_body)
    return call(idx, tpad)


def kernel(indices, table):
    idx = indices.astype(jnp.int32).reshape(_NUM_WORKERS, _N_CHUNKS, _CHUNK)
    tpad = jnp.pad(table, ((0, _VPAD - _VOCAB), (0, _WPAD - _DIM)))
    out = _sc_gather(idx, tpad)
    return out.reshape(_BATCH, _WPAD)[:, :_DIM]


# speedup vs baseline: 12.7986x; 1.4110x over previous
import functools

import jax
import jax.numpy as jnp
from jax import lax
from jax.experimental import pallas as pl
from jax.experimental.pallas import tpu as pltpu
from jax.experimental.pallas import tpu_sc as plsc

_NUM_CORES = 2
_NUM_SUBCORES = 16
_NUM_WORKERS = _NUM_CORES * _NUM_SUBCORES

_BATCH = 16384
_DIM = 64

_ROWS_PER_W = _BATCH // _NUM_WORKERS
_WAVE = 16
_N_WAVES = _ROWS_PER_W // _WAVE
_LANES = 16


def _gather_body(idx_hbm, table_hbm, out_hbm, idx_v, ring_v, dst_v, sem0, sem1):
    wid = lax.axis_index("s") * _NUM_CORES + lax.axis_index("c")
    pltpu.sync_copy(idx_hbm.at[wid], idx_v)

    def fire(w, sem):
        iv = idx_v[w, :]
        for j in range(_WAVE):
            i = iv[j]
            t8 = (i >> 3) * 8
            pltpu.async_copy(
                table_hbm.at[pl.ds(t8, 8), :],
                ring_v.at[(w & 1) * _WAVE + j],
                sem,
            )

    def drain(sem):
        for _ in range(_WAVE):
            pltpu.make_async_copy(
                table_hbm.at[pl.ds(0, 8), :], ring_v.at[0], sem
            ).wait()

    def extract(w):
        iv = idx_v[w, :]
        for j in range(_WAVE):
            i = iv[j]
            r = i & 7
            slot = (w & 1) * _WAVE + j
            row = w * _WAVE + j
            for q in range(_DIM // _LANES):
                dst_v[row, pl.ds(q * _LANES, _LANES)] = ring_v[
                    slot, r, pl.ds(q * _LANES, _LANES)
                ]

    fire(0, sem0)

    def body(w, carry):
        @pl.when(jnp.logical_and(w + 1 < _N_WAVES, (w & 1) == 0))
        def _():
            fire(w + 1, sem1)

        @pl.when(jnp.logical_and(w + 1 < _N_WAVES, (w & 1) == 1))
        def _():
            fire(w + 1, sem0)

        @pl.when((w & 1) == 0)
        def _():
            drain(sem0)

        @pl.when((w & 1) == 1)
        def _():
            drain(sem1)

        extract(w)
        return carry

    lax.fori_loop(0, _N_WAVES, body, 0)
    pltpu.sync_copy(dst_v, out_hbm.at[wid])


@jax.jit
def _sc_gather(idx, table):
    mesh = plsc.VectorSubcoreMesh(core_axis_name="c", subcore_axis_name="s")
    call = functools.partial(
        pl.kernel,
        mesh=mesh,
        out_type=jax.ShapeDtypeStruct(
            (_NUM_WORKERS, _ROWS_PER_W, _DIM), jnp.float32
        ),
        scratch_types=[
            pltpu.VMEM((_N_WAVES, _WAVE), jnp.int32),
            pltpu.VMEM((2 * _WAVE, 8, _DIM), jnp.float32),
            pltpu.VMEM((_ROWS_PER_W, _DIM), jnp.float32),
            pltpu.SemaphoreType.DMA,
            pltpu.SemaphoreType.DMA,
        ],
        compiler_params=pltpu.CompilerParams(use_tc_tiling_on_sc=True),
    )(_gather_body)
    return call(idx, table)


def kernel(indices, table):
    idx = indices.astype(jnp.int32).reshape(_NUM_WORKERS, _N_WAVES, _WAVE)
    out = _sc_gather(idx, table)
    return out.reshape(_BATCH, _DIM)


# speedup vs baseline: 17.6662x; 1.3803x over previous
import functools

import jax
import jax.numpy as jnp
from jax import lax
from jax.experimental import pallas as pl
from jax.experimental.pallas import tpu as pltpu
from jax.experimental.pallas import tpu_sc as plsc

_NUM_CORES = 2
_NUM_SUBCORES = 16
_NUM_WORKERS = _NUM_CORES * _NUM_SUBCORES

_BATCH = 16384
_DIM = 64
_CBLK = 128

_ROWS_PER_W = _BATCH // _NUM_WORKERS
_WAVE = 2
_N_WAVES = _ROWS_PER_W // _WAVE
_LANES = 16


def _gather_body(idx_hbm, tt_hbm, out_hbm, idx_v, ring_v, dst_v, sem0, sem1):
    wid = lax.axis_index("s") * _NUM_CORES + lax.axis_index("c")
    pltpu.sync_copy(idx_hbm.at[wid], idx_v)

    def fire(w, sem):
        iv = idx_v[w, :]
        for j in range(_WAVE):
            i = iv[j]
            col0 = (i >> 7) * _CBLK
            pltpu.async_copy(
                tt_hbm.at[:, pl.ds(col0, _CBLK)],
                ring_v.at[(w & 1) * _WAVE + j],
                sem,
            )

    def drain(sem):
        for _ in range(_WAVE):
            pltpu.make_async_copy(
                tt_hbm.at[:, pl.ds(0, _CBLK)], ring_v.at[0], sem
            ).wait()

    def extract(w):
        iv = idx_v[w, :]
        for j in range(_WAVE):
            i = iv[j]
            d = i & (_CBLK - 1)
            slot = (w & 1) * _WAVE + j
            row = w * _WAVE + j
            cols = jnp.full((_LANES,), d, jnp.int32)
            for q in range(_DIM // _LANES):
                rows = lax.iota(jnp.int32, _LANES) + q * _LANES
                v = plsc.load_gather(ring_v.at[slot], [rows, cols])
                dst_v[row, pl.ds(q * _LANES, _LANES)] = v

    fire(0, sem0)

    def body(w, carry):
        @pl.when(jnp.logical_and(w + 1 < _N_WAVES, (w & 1) == 0))
        def _():
            fire(w + 1, sem1)

        @pl.when(jnp.logical_and(w + 1 < _N_WAVES, (w & 1) == 1))
        def _():
            fire(w + 1, sem0)

        @pl.when((w & 1) == 0)
        def _():
            drain(sem0)

        @pl.when((w & 1) == 1)
        def _():
            drain(sem1)

        extract(w)
        return carry

    lax.fori_loop(0, _N_WAVES, body, 0)
    pltpu.sync_copy(dst_v, out_hbm.at[wid])


@jax.jit
def _sc_gather(idx, tt):
    mesh = plsc.VectorSubcoreMesh(core_axis_name="c", subcore_axis_name="s")
    call = functools.partial(
        pl.kernel,
        mesh=mesh,
        out_type=jax.ShapeDtypeStruct(
            (_NUM_WORKERS, _ROWS_PER_W, _DIM), jnp.float32
        ),
        scratch_types=[
            pltpu.VMEM((_N_WAVES, _LANES), jnp.int32),
            pltpu.VMEM((2 * _WAVE, _DIM, _CBLK), jnp.float32),
            pltpu.VMEM((_ROWS_PER_W, _DIM), jnp.float32),
            pltpu.SemaphoreType.DMA,
            pltpu.SemaphoreType.DMA,
        ],
        compiler_params=pltpu.CompilerParams(
            use_tc_tiling_on_sc=True, needs_layout_passes=False
        ),
    )(_gather_body)
    return call(idx, tt)


def kernel(indices, table):
    idx = indices.astype(jnp.int32).reshape(_NUM_WORKERS, _N_WAVES, _WAVE)
    idx = jnp.pad(idx, ((0, 0), (0, 0), (0, _LANES - _WAVE)))
    out = _sc_gather(idx, table.T)
    return out.reshape(_BATCH, _DIM)
